# baseline (speedup 1.0000x reference)
import jax
import jax.numpy as jnp
import numpy as np
from jax.experimental import pallas as pl
from jax.experimental.pallas import tpu as pltpu

_VMEM_LIMIT = 48 * 1024 * 1024
_CHUNK = 16


def _interp_mat(out_size, in_size):
    m = np.zeros((out_size, in_size), np.float32)
    for o in range(out_size):
        src = o * (in_size - 1) / (out_size - 1) if out_size > 1 else 0.0
        lo = int(np.floor(src))
        hi = min(lo + 1, in_size - 1)
        a = src - lo
        m[o, lo] += 1.0 - a
        m[o, hi] += a
    return jnp.asarray(m)


def _store_shifted(pw_ref, x, r0, Ctot, c0):
    rows, W, Cx = x.shape
    h0 = 1 + r0
    pw_ref[h0:h0 + rows, 1:W, c0:c0 + Cx] = x[:, 0:W - 1]
    pw_ref[h0:h0 + rows, 0:W, Ctot + c0:Ctot + c0 + Cx] = x
    pw_ref[h0:h0 + rows, 0:W - 1, 2 * Ctot + c0:2 * Ctot + c0 + Cx] = x[:, 1:W]


def _zero_plane_border(pw_ref, W, C, c0, Ctot):
    Hp = pw_ref.shape[0]
    pw_ref[0:1, :, c0:c0 + C] = jnp.zeros((1, W, C), pw_ref.dtype)
    pw_ref[0:1, :, Ctot + c0:Ctot + c0 + C] = jnp.zeros((1, W, C),
                                                        pw_ref.dtype)
    pw_ref[0:1, :, 2 * Ctot + c0:2 * Ctot + c0 + C] = jnp.zeros(
        (1, W, C), pw_ref.dtype)
    pw_ref[Hp - 1:Hp, :, c0:c0 + C] = jnp.zeros((1, W, C), pw_ref.dtype)
    pw_ref[Hp - 1:Hp, :, Ctot + c0:Ctot + c0 + C] = jnp.zeros(
        (1, W, C), pw_ref.dtype)
    pw_ref[Hp - 1:Hp, :, 2 * Ctot + c0:2 * Ctot + c0 + C] = jnp.zeros(
        (1, W, C), pw_ref.dtype)
    pw_ref[:, 0:1, c0:c0 + C] = jnp.zeros((Hp, 1, C), pw_ref.dtype)
    pw_ref[:, W - 1:W, 2 * Ctot + c0:2 * Ctot + c0 + C] = jnp.zeros(
        (Hp, 1, C), pw_ref.dtype)


def _conv_rows(pw_ref, w3, b, r0, ch, W):
    K3 = pw_ref.shape[-1]
    acc = jnp.dot(pw_ref[r0:r0 + ch].reshape(ch * W, K3), w3[0],
                  preferred_element_type=jnp.float32)
    acc = acc + jnp.dot(pw_ref[r0 + 1:r0 + 1 + ch].reshape(ch * W, K3), w3[1],
                        preferred_element_type=jnp.float32)
    acc = acc + jnp.dot(pw_ref[r0 + 2:r0 + 2 + ch].reshape(ch * W, K3), w3[2],
                        preferred_element_type=jnp.float32)
    return jnp.maximum(acc + b, 0.0)


def _up_block_kernel(x1_ref, x2_ref, wh_ref, wwt_ref,
                     w1_ref, s1_ref, b1_ref, w2_ref, s2_ref, b2_ref, o_ref,
                     pw1_ref, pw2_ref):
    _, H1, W1, C1 = x1_ref.shape
    _, H2, W2, C2 = x2_ref.shape
    Cin = C1 + C2
    Cmid = w1_ref.shape[-1]
    Cout = w2_ref.shape[-1]
    bf16 = jnp.bfloat16
    ch = _CHUNK if H2 % _CHUNK == 0 else H2

    w1f = (w1_ref[...] * s1_ref[0][None, None, None, :]).astype(bf16)
    w1f = w1f.reshape(3, 3 * Cin, Cmid)
    w2f = (w2_ref[...] * s2_ref[0][None, None, None, :]).astype(bf16)
    w2f = w2f.reshape(3, 3 * Cmid, Cout)
    b1 = b1_ref[...]
    b2 = b2_ref[...]

    x1hwc = x1_ref[0].reshape(H1, W1 * C1)
    t = jnp.dot(wh_ref[...], x1hwc,
                preferred_element_type=jnp.float32)
    t = jnp.transpose(t.reshape(H2, W1, C1), (0, 2, 1))
    u = jnp.dot(t.reshape(H2 * C1, W1), wwt_ref[...],
                preferred_element_type=jnp.float32)
    up = jnp.transpose(u.reshape(H2, C1, W2), (0, 2, 1))

    _zero_plane_border(pw1_ref, W2, C2, 0, Cin)
    _zero_plane_border(pw1_ref, W2, C1, C2, Cin)
    _store_shifted(pw1_ref, x2_ref[0].astype(bf16), 0, Cin, 0)
    _store_shifted(pw1_ref, up.astype(bf16), 0, Cin, C2)

    _zero_plane_border(pw2_ref, W2, Cmid, 0, Cmid)
    for r0 in range(0, H2, ch):
        m = _conv_rows(pw1_ref, w1f, b1, r0, ch, W2)
        _store_shifted(pw2_ref, m.astype(bf16).reshape(ch, W2, Cmid),
                       r0, Cmid, 0)

    for r0 in range(0, H2, ch):
        y = _conv_rows(pw2_ref, w2f, b2, r0, ch, W2)
        o_ref[0, r0:r0 + ch] = y.reshape(ch, W2, Cout)


def kernel(x1_nchw, x2_nchw, w1, s1, b1, w2, s2, b2):
    N, C1, H1, W1 = x1_nchw.shape
    _, C2, H2, W2 = x2_nchw.shape
    Cin = C1 + C2
    Cmid = w1.shape[-1]
    Cout = w2.shape[-1]

    b1r = b1.reshape(1, Cmid).astype(jnp.float32)
    b2r = b2.reshape(1, Cout).astype(jnp.float32)
    s1r = s1.reshape(1, Cmid).astype(jnp.float32)
    s2r = s2.reshape(1, Cout).astype(jnp.float32)

    wh = _interp_mat(H2, H1)
    wwt = _interp_mat(W2, W1).T

    x1h = jnp.transpose(x1_nchw, (0, 2, 3, 1))
    x2h = jnp.transpose(x2_nchw, (0, 2, 3, 1))

    yh = pl.pallas_call(
        _up_block_kernel,
        out_shape=jax.ShapeDtypeStruct((N, H2, W2, Cout), x2_nchw.dtype),
        grid=(N,),
        in_specs=[
            pl.BlockSpec((1, H1, W1, C1), lambda n: (n, 0, 0, 0)),
            pl.BlockSpec((1, H2, W2, C2), lambda n: (n, 0, 0, 0)),
            pl.BlockSpec((H2, H1), lambda n: (0, 0)),
            pl.BlockSpec((W1, W2), lambda n: (0, 0)),
            pl.BlockSpec((3, 3, Cin, Cmid), lambda n: (0, 0, 0, 0)),
            pl.BlockSpec((1, Cmid), lambda n: (0, 0)),
            pl.BlockSpec((1, Cmid), lambda n: (0, 0)),
            pl.BlockSpec((3, 3, Cmid, Cout), lambda n: (0, 0, 0, 0)),
            pl.BlockSpec((1, Cout), lambda n: (0, 0)),
            pl.BlockSpec((1, Cout), lambda n: (0, 0)),
        ],
        out_specs=pl.BlockSpec((1, H2, W2, Cout), lambda n: (n, 0, 0, 0)),
        scratch_shapes=[
            pltpu.VMEM((H2 + 2, W2, 3 * Cin), jnp.bfloat16),
            pltpu.VMEM((H2 + 2, W2, 3 * Cmid), jnp.bfloat16),
        ],
        compiler_params=pltpu.CompilerParams(
            dimension_semantics=("parallel",),
            vmem_limit_bytes=_VMEM_LIMIT),
    )(x1h, x2h, wh, wwt, w1, s1r, b1r, w2, s2r, b2r)
    return jnp.transpose(yh, (0, 3, 1, 2))
